# baseline (speedup 1.0000x reference)
import jax
import jax.numpy as jnp
from jax.experimental import pallas as pl
from jax.experimental.pallas import tpu as pltpu

_E = 16
_C = 320
_D = 1024
_N = 4096
_B = 1024
_ES = 4
_NJ = _E // _ES


def _router_body(x_ref, w_ref, disp_ref, comb_ref, counts_ref):
    j = pl.program_id(1)

    @pl.when((pl.program_id(0) == 0) & (j == 0))
    def _init():
        counts_ref[...] = jnp.zeros_like(counts_ref)

    lg = jax.lax.dot_general(
        w_ref[...], x_ref[...], (((0,), (1,)), ((), ())),
        preferred_element_type=jnp.float32)
    m = jnp.max(lg, axis=0, keepdims=True)
    e = jnp.exp(lg - m)
    probs = e / jnp.sum(e, axis=0, keepdims=True)
    gate = jnp.max(probs, axis=0, keepdims=True)
    iota_e = jax.lax.broadcasted_iota(jnp.int32, (_E, _B), 0)
    expert = jnp.min(jnp.where(probs == gate, iota_e, _E), axis=0,
                     keepdims=True)
    mask = (iota_e == expert).astype(jnp.float32)

    r = jax.lax.broadcasted_iota(jnp.int32, (_B, _B), 0)
    c = jax.lax.broadcasted_iota(jnp.int32, (_B, _B), 1)
    triu = (r <= c).astype(jnp.float32)
    csum = jnp.dot(mask, triu, preferred_element_type=jnp.float32)
    pos = csum + counts_ref[...]
    @pl.when(j == _NJ - 1)
    def _advance():
        counts_ref[...] = counts_ref[...] + csum[:, _B - 1 : _B]

    p_sel = jnp.where((mask > 0.0) & (pos < float(_C)), pos, -1.0)

    iota_c = jax.lax.broadcasted_iota(jnp.int32, (_C, 1), 0).astype(jnp.float32)
    for es in range(_ES):
        sel = (iota_e == j * _ES + es).astype(jnp.float32)
        row = jnp.sum(p_sel * sel, axis=0, keepdims=True)
        d = (iota_c == row).astype(jnp.float32)
        disp_ref[es] = d
        comb_ref[es] = d * gate


def kernel(inputs, W):
    disp_t, comb_t = pl.pallas_call(
        _router_body,
        grid=(_N // _B, _NJ),
        in_specs=[
            pl.BlockSpec((_B, _D), lambda i, j: (i, 0)),
            pl.BlockSpec((_D, _E), lambda i, j: (0, 0)),
        ],
        out_specs=[
            pl.BlockSpec((_ES, _C, _B), lambda i, j: (j, 0, i)),
            pl.BlockSpec((_ES, _C, _B), lambda i, j: (j, 0, i)),
        ],
        out_shape=[
            jax.ShapeDtypeStruct((_E, _C, _N), jnp.float32),
            jax.ShapeDtypeStruct((_E, _C, _N), jnp.float32),
        ],
        scratch_shapes=[pltpu.VMEM((_E, 1), jnp.float32)],
        compiler_params=pltpu.CompilerParams(
            dimension_semantics=("arbitrary", "arbitrary")
        ),
    )(inputs, W)
    return disp_t.transpose(2, 0, 1), comb_t.transpose(2, 0, 1)


# speedup vs baseline: 1.0509x; 1.0509x over previous
import jax
import jax.numpy as jnp
from jax.experimental import pallas as pl
from jax.experimental.pallas import tpu as pltpu

_E = 16
_C = 320
_D = 1024
_N = 4096
_B = 512


def _router_body(x_ref, w_ref, disp_ref, comb_ref, counts_ref):
    @pl.when(pl.program_id(0) == 0)
    def _init():
        counts_ref[...] = jnp.zeros_like(counts_ref)

    lg = jax.lax.dot_general(
        w_ref[...], x_ref[...], (((0,), (1,)), ((), ())),
        preferred_element_type=jnp.float32)
    m = jnp.max(lg, axis=0, keepdims=True)
    e = jnp.exp(lg - m)
    probs = e / jnp.sum(e, axis=0, keepdims=True)
    gate = jnp.max(probs, axis=0, keepdims=True)
    iota_e = jax.lax.broadcasted_iota(jnp.int32, (_E, _B), 0)
    expert = jnp.min(jnp.where(probs == gate, iota_e, _E), axis=0,
                     keepdims=True)
    mask = (iota_e == expert).astype(jnp.float32)

    r = jax.lax.broadcasted_iota(jnp.int32, (_B, _B), 0)
    c = jax.lax.broadcasted_iota(jnp.int32, (_B, _B), 1)
    triu = (r <= c).astype(jnp.float32)
    csum = jnp.dot(mask, triu, preferred_element_type=jnp.float32)
    pos = csum + counts_ref[...]
    counts_ref[...] = counts_ref[...] + csum[:, _B - 1 : _B]
    p_sel = jnp.where((mask > 0.0) & (pos < float(_C)), pos, -1.0)

    iota_c = jax.lax.broadcasted_iota(jnp.int32, (_C, 1), 0).astype(jnp.float32)
    for ex in range(_E):
        row = p_sel[ex : ex + 1, :]
        d = (iota_c == row).astype(jnp.float32)
        disp_ref[ex] = d
        comb_ref[ex] = d * gate


def kernel(inputs, W):
    disp_t, comb_t = pl.pallas_call(
        _router_body,
        grid=(_N // _B,),
        in_specs=[
            pl.BlockSpec((_B, _D), lambda i: (i, 0)),
            pl.BlockSpec((_D, _E), lambda i: (0, 0)),
        ],
        out_specs=[
            pl.BlockSpec((_E, _C, _B), lambda i: (0, 0, i)),
            pl.BlockSpec((_E, _C, _B), lambda i: (0, 0, i)),
        ],
        out_shape=[
            jax.ShapeDtypeStruct((_E, _C, _N), jnp.float32),
            jax.ShapeDtypeStruct((_E, _C, _N), jnp.float32),
        ],
        scratch_shapes=[pltpu.VMEM((_E, 1), jnp.float32)],
        compiler_params=pltpu.CompilerParams(
            dimension_semantics=("arbitrary",)
        ),
    )(inputs, W)
    return disp_t.transpose(2, 0, 1), comb_t.transpose(2, 0, 1)
